# Initial kernel scaffold; baseline (speedup 1.0000x reference)
import jax
import jax.numpy as jnp
from jax.experimental import pallas as pl


def kernel(x, wpe):
    raise NotImplementedError("write your pallas kernel here")


# speedup vs baseline: 2.0867x; 2.0867x over previous
import jax
import jax.numpy as jnp
from jax.experimental import pallas as pl

BS = 512


def _posenc_body(x_ref, w_ref, o_ref):
    o_ref[...] = x_ref[...] + w_ref[...]


def kernel(x, wpe):
    B, S, D = x.shape
    grid = (S // BS,)
    return pl.pallas_call(
        _posenc_body,
        grid=grid,
        in_specs=[
            pl.BlockSpec((B, BS, D), lambda i: (0, i, 0)),
            pl.BlockSpec((BS, D), lambda i: (i, 0)),
        ],
        out_specs=pl.BlockSpec((B, BS, D), lambda i: (0, i, 0)),
        out_shape=jax.ShapeDtypeStruct((B, S, D), x.dtype),
    )(x, wpe)
